# baseline (speedup 1.0000x reference)
import functools

import jax
import jax.numpy as jnp
from jax import lax
from jax.experimental import pallas as pl
from jax.experimental.pallas import tpu as pltpu
from jax.experimental.pallas import tpu_sc as plsc

N = 100000
E = 1600000
DW = 16
BLK = 128
NBLKP = 12512
EPAD = NBLKP * BLK - E
BLK_PER_TILE = NBLKP // 32
NP_ACC = 100096
ROWS_PER_TILE = NP_ACC // 16
CH = 6
NCH = 65


@functools.cache
def _sc_kernel():
    mesh = plsc.VectorSubcoreMesh(core_axis_name="c", subcore_axis_name="s",
                                  num_cores=2, num_subcores=16)
    return functools.partial(
        pl.kernel,
        out_type=jax.ShapeDtypeStruct((2 * NP_ACC, DW), jnp.float32),
        mesh=mesh,
        scratch_types=[
            pltpu.VMEM_SHARED((NP_ACC, DW), jnp.float32),
            pltpu.VMEM((2 * CH, BLK), jnp.int32),
            pltpu.VMEM((2 * CH, BLK), jnp.int32),
            pltpu.VMEM((2 * CH, BLK, DW), jnp.float32),
            pltpu.SemaphoreType.DMA,
            pltpu.SemaphoreType.DMA,
        ],
        compiler_params=pltpu.CompilerParams(use_tc_tiling_on_sc=False),
    )(_sc_body)


def _sc_segment_sum(tab, srcb, dstb):
    return _sc_kernel()(tab, srcb, dstb).reshape(2 * NP_ACC * DW // 128, 128)


def _sc_body(tab, srcb, dstb, out, acc, sidx, didx, rows, gsem, ssem):
    c = lax.axis_index("c")
    s = lax.axis_index("s")
    wid = c * 16 + s

    z16 = jnp.zeros((16,), jnp.float32)

    def _zb(i, carry):
        rows[0, i, pl.ds(0, 16)] = z16
        return carry

    lax.fori_loop(0, BLK, _zb, 0)
    zbase = s * ROWS_PER_TILE
    for k in range(ROWS_PER_TILE // BLK):
        pltpu.sync_copy(rows.at[0], acc.at[pl.ds(zbase + k * BLK, BLK)])
    pltpu.sync_copy(rows.at[0], acc.at[pl.ds(zbase + ROWS_PER_TILE - BLK, BLK)])
    plsc.subcore_barrier()

    start = wid * BLK_PER_TILE

    def _stage(t, h):
        base = start + t * CH
        pltpu.sync_copy(srcb.at[pl.ds(base, CH)], sidx.at[pl.ds(h * CH, CH)])
        pltpu.sync_copy(dstb.at[pl.ds(base, CH)], didx.at[pl.ds(h * CH, CH)])

    def _fire_gathers(h):
        for j in range(CH):
            pltpu.async_copy(tab.at[sidx.at[h * CH + j]],
                             rows.at[h * CH + j], gsem)

    def _drain_gathers(h):
        for j in range(CH):
            pltpu.make_async_copy(tab.at[sidx.at[h * CH + j]],
                                  rows.at[h * CH + j], gsem).wait()

    _stage(0, 0)
    _fire_gathers(0)

    def _chunk(t, carry):
        cur = t % 2
        nxt = 1 - cur
        _drain_gathers(cur)

        @pl.when(t < NCH - 1)
        def _prefetch():
            _stage(t + 1, nxt)
            _fire_gathers(nxt)

        sd = [pltpu.async_copy(rows.at[cur * CH + j],
                               acc.at[didx.at[cur * CH + j]],
                               ssem, add=True) for j in range(CH)]
        for d in sd:
            d.wait()
        return carry

    lax.fori_loop(0, NCH, _chunk, 0)
    tbase = start + CH * NCH
    pltpu.sync_copy(srcb.at[pl.ds(tbase, 1)], sidx.at[pl.ds(0, 1)])
    pltpu.sync_copy(dstb.at[pl.ds(tbase, 1)], didx.at[pl.ds(0, 1)])
    pltpu.async_copy(tab.at[sidx.at[0]], rows.at[0], gsem).wait()
    pltpu.sync_copy(rows.at[0], acc.at[didx.at[0]], add=True)
    plsc.subcore_barrier()

    pltpu.sync_copy(acc.at[pl.ds(zbase, ROWS_PER_TILE)],
                    out.at[pl.ds(c * NP_ACC + zbase, ROWS_PER_TILE)])


MROWS = NP_ACC * DW // 128
_RC = MROWS // 4


def _combine_body(pa0_ref, pa1_ref, pb0_ref, pb1_ref, s_ref, oa_ref, ob_ref):
    sa = pa0_ref[...] + pa1_ref[...]
    sb = pb0_ref[...] + pb1_ref[...]
    degb = jnp.maximum(
        lax.dot_general(sb, s_ref[...], (((1,), (0,)), ((), ())),
                        preferred_element_type=jnp.float32), 1.0)
    oa_ref[...] = sa / degb
    lane = lax.broadcasted_iota(jnp.int32, (_RC, 128), 1) % DW
    ob_ref[...] = jnp.where(lane == 0, sb / degb,
                            jnp.where(lane == 1, 1.0, 0.0))


def _combine(partsA, partsB, S):
    return pl.pallas_call(
        _combine_body,
        grid=(4,),
        in_specs=[
            pl.BlockSpec((_RC, 128), lambda i: (i, 0)),
            pl.BlockSpec((_RC, 128), lambda i: (i + 4, 0)),
            pl.BlockSpec((_RC, 128), lambda i: (i, 0)),
            pl.BlockSpec((_RC, 128), lambda i: (i + 4, 0)),
            pl.BlockSpec((128, 128), lambda i: (0, 0)),
        ],
        out_specs=[pl.BlockSpec((_RC, 128), lambda i: (i, 0))] * 2,
        out_shape=[jax.ShapeDtypeStruct((MROWS, 128), jnp.float32)] * 2,
    )(partsA, partsA, partsB, partsB, S)


_RF = MROWS // 23


def _final_body(xv_ref, ta_ref, tb_ref, pa0_ref, pa1_ref, pb0_ref, pb1_ref,
                s_ref, abd_ref, vmat_ref, dbd_ref, w3_ref, b3_ref, w4_ref,
                b4_ref, w5_ref, b5_ref, o_ref):
    f32 = jnp.float32
    dn = (((1,), (0,)), ((), ()))
    dot = lambda a, b: lax.dot_general(a, b, dn, preferred_element_type=f32)
    sa = pa0_ref[...] + pa1_ref[...]
    sb = pb0_ref[...] + pb1_ref[...]
    degr = dot(sb, s_ref[...])
    degb = jnp.maximum(degr, 1.0)
    vin = jnp.concatenate(
        [ta_ref[...], tb_ref[...], sa / degb, sb / degb,
         jnp.where(degr > 0.0, 1.0, 0.0)], axis=1)
    h2 = dot(xv_ref[...], abd_ref[...]) + dot(vin, vmat_ref[...]) + dbd_ref[0:1]
    h3 = jnp.maximum(dot(h2, w3_ref[...]) + b3_ref[0:1], 0.0)
    h4 = jnp.maximum(dot(h3, w4_ref[...]) + b4_ref[0:1], 0.0)
    o_ref[...] = jax.nn.sigmoid(dot(h4, w5_ref[...]) + b5_ref[0:1])


def _final(xv, tA, tB, pA, pB, S, abd, vmat, dbd, w3bd, b3bd, w4bd, b4bd,
           w5bd, b5bd):
    full = lambda shape: pl.BlockSpec(shape, lambda i: tuple(0 for _ in shape))
    mspec = lambda off: pl.BlockSpec((_RF, 128), lambda i, o=off: (i + o, 0))
    return pl.pallas_call(
        _final_body,
        grid=(MROWS // _RF,),
        in_specs=[
            pl.BlockSpec((_RF, 256), lambda i: (i, 0)),
            mspec(0),
            mspec(0),
            mspec(0),
            mspec(23),
            mspec(0),
            mspec(23),
            full((128, 128)),
            full((256, 512)),
            full((640, 512)),
            full((8, 512)),
            full((512, 256)),
            full((8, 256)),
            full((256, 128)),
            full((8, 128)),
            full((128, 8)),
            full((8, 8)),
        ],
        out_specs=pl.BlockSpec((_RF, 8), lambda i: (i, 0)),
        out_shape=jax.ShapeDtypeStruct((MROWS, 8), jnp.float32),
    )(xv, tA, tB, pA, pA, pB, pB, S, abd, vmat, dbd, w3bd, b3bd, w4bd, b4bd,
      w5bd, b5bd)


def kernel(x, edge_index, W1_self, W1_neigh, b1, W2_self, W2_neigh, b2,
           W3, b3, W4, b4, W5, b5):
    f32 = jnp.float32
    zpad = jnp.zeros((NP_ACC - N, DW), f32)
    tabA = jnp.concatenate([x[:, 0:16], zpad], axis=0)
    tabB = jnp.concatenate([
        jnp.concatenate([x[:, 16:17], jnp.ones((N, 1), f32),
                         jnp.zeros((N, DW - 2), f32)], axis=1), zpad], axis=0)

    pad = jnp.full((EPAD,), N, jnp.int32)
    srcp = jnp.concatenate([edge_index[0], pad]).reshape(NBLKP, BLK)
    dstp = jnp.concatenate([edge_index[1], pad]).reshape(NBLKP, BLK)

    from jax.scipy.linalg import block_diag
    A = W1_self @ W2_self
    B = W1_neigh @ W2_self + W1_self @ W2_neigh
    C = W1_neigh @ W2_neigh
    d0 = b1 @ W2_self + b2
    c0 = b1 @ W2_neigh
    A32 = jnp.concatenate([A, jnp.zeros((15, 64), f32)], axis=0)
    row1 = lambda v: jnp.zeros((16, 64), f32).at[0].set(v)
    bd8 = lambda m: block_diag(*([m] * 8))
    abd = bd8(A32)
    vmat = jnp.concatenate(
        [bd8(B[:16]), bd8(row1(B[16])), bd8(C[:16]), bd8(row1(C[16])),
         bd8(row1(c0))], axis=0)
    rowp = lambda v, w: jnp.zeros((8, w), f32).at[0].set(jnp.tile(v, 8))
    dbd = rowp(d0, 512)
    w3bd, b3bd = bd8(W3), rowp(b3, 256)
    w4bd, b4bd = bd8(W4), rowp(b4, 128)
    w5bd, b5bd = bd8(W5), rowp(b5, 8)

    li = jnp.arange(128)
    S = ((li[:, None] % DW == 1) & (li[:, None] // DW == li[None, :] // DW)
         ).astype(f32)
    xv = jnp.concatenate(
        [x, jnp.zeros((N, 15), f32)], axis=1)
    xv = jnp.concatenate([xv, jnp.zeros((NP_ACC - N, 32), f32)], axis=0)
    xv = xv.reshape(MROWS, 256)

    partsA1 = _sc_segment_sum(tabA, srcp, dstp)
    partsB1 = _sc_segment_sum(tabB, srcp, dstp)
    t2A, t2B = _combine(partsA1, partsB1, S)
    partsA2 = _sc_segment_sum(t2A.reshape(NP_ACC, DW), srcp, dstp)
    partsB2 = _sc_segment_sum(t2B.reshape(NP_ACC, DW), srcp, dstp)
    out8 = _final(xv, t2A, t2B, partsA2, partsB2, S, abd, vmat, dbd,
                  w3bd, b3bd, w4bd, b4bd, w5bd, b5bd)
    return out8.reshape(NP_ACC, 1)[:N]


# speedup vs baseline: 1.0999x; 1.0999x over previous
import functools

import jax
import jax.numpy as jnp
from jax import lax
from jax.experimental import pallas as pl
from jax.experimental.pallas import tpu as pltpu
from jax.experimental.pallas import tpu_sc as plsc

N = 100000
E = 1600000
DW = 16
BLK = 128
NBLKP = 12512
EPAD = NBLKP * BLK - E
BLK_PER_TILE = NBLKP // 32
NP_ACC = 100096
ROWS_PER_TILE = NP_ACC // 16
CH = 13
NCH = 30


@functools.cache
def _sc_kernel():
    mesh = plsc.VectorSubcoreMesh(core_axis_name="c", subcore_axis_name="s",
                                  num_cores=2, num_subcores=16)
    return functools.partial(
        pl.kernel,
        out_type=jax.ShapeDtypeStruct((2 * NP_ACC, DW), jnp.float32),
        mesh=mesh,
        scratch_types=[
            pltpu.VMEM_SHARED((NP_ACC, DW), jnp.float32),
            pltpu.VMEM((CH, BLK), jnp.int32),
            pltpu.VMEM((CH, BLK), jnp.int32),
            pltpu.VMEM((CH, BLK, DW), jnp.float32),
            pltpu.SemaphoreType.DMA,
            pltpu.SemaphoreType.DMA,
        ],
        compiler_params=pltpu.CompilerParams(use_tc_tiling_on_sc=False),
    )(_sc_body)


def _sc_segment_sum(tab, srcb, dstb):
    return _sc_kernel()(tab, srcb, dstb).reshape(2 * NP_ACC * DW // 128, 128)


def _sc_body(tab, srcb, dstb, out, acc, sidx, didx, rows, gsem, ssem):
    c = lax.axis_index("c")
    s = lax.axis_index("s")
    wid = c * 16 + s

    z16 = jnp.zeros((16,), jnp.float32)

    def _zb(i, carry):
        rows[0, i, pl.ds(0, 16)] = z16
        return carry

    lax.fori_loop(0, BLK, _zb, 0)
    zbase = s * ROWS_PER_TILE
    for k in range(ROWS_PER_TILE // BLK):
        pltpu.sync_copy(rows.at[0], acc.at[pl.ds(zbase + k * BLK, BLK)])
    pltpu.sync_copy(rows.at[0], acc.at[pl.ds(zbase + ROWS_PER_TILE - BLK, BLK)])
    plsc.subcore_barrier()

    start = wid * BLK_PER_TILE

    def _chunk(t, carry):
        base = start + t * CH
        pltpu.sync_copy(srcb.at[pl.ds(base, CH)], sidx)
        pltpu.sync_copy(dstb.at[pl.ds(base, CH)], didx)
        gd = [pltpu.async_copy(tab.at[sidx.at[j]], rows.at[j], gsem)
              for j in range(CH)]
        for d in gd:
            d.wait()
        sd = [pltpu.async_copy(rows.at[j], acc.at[didx.at[j]], ssem, add=True)
              for j in range(CH)]
        for d in sd:
            d.wait()
        return carry

    lax.fori_loop(0, NCH, _chunk, 0)
    tbase = start + CH * NCH
    pltpu.sync_copy(srcb.at[pl.ds(tbase, 1)], sidx.at[pl.ds(0, 1)])
    pltpu.sync_copy(dstb.at[pl.ds(tbase, 1)], didx.at[pl.ds(0, 1)])
    pltpu.async_copy(tab.at[sidx.at[0]], rows.at[0], gsem).wait()
    pltpu.sync_copy(rows.at[0], acc.at[didx.at[0]], add=True)
    plsc.subcore_barrier()

    pltpu.sync_copy(acc.at[pl.ds(zbase, ROWS_PER_TILE)],
                    out.at[pl.ds(c * NP_ACC + zbase, ROWS_PER_TILE)])


MROWS = NP_ACC * DW // 128
_RC = MROWS // 4


def _combine_body(pa0_ref, pa1_ref, pb0_ref, pb1_ref, s_ref, oa_ref, ob_ref):
    sa = pa0_ref[...] + pa1_ref[...]
    sb = pb0_ref[...] + pb1_ref[...]
    degb = jnp.maximum(
        lax.dot_general(sb, s_ref[...], (((1,), (0,)), ((), ())),
                        preferred_element_type=jnp.float32), 1.0)
    oa_ref[...] = sa / degb
    lane = lax.broadcasted_iota(jnp.int32, (_RC, 128), 1) % DW
    ob_ref[...] = jnp.where(lane == 0, sb / degb,
                            jnp.where(lane == 1, 1.0, 0.0))


def _combine(partsA, partsB, S):
    return pl.pallas_call(
        _combine_body,
        grid=(4,),
        in_specs=[
            pl.BlockSpec((_RC, 128), lambda i: (i, 0)),
            pl.BlockSpec((_RC, 128), lambda i: (i + 4, 0)),
            pl.BlockSpec((_RC, 128), lambda i: (i, 0)),
            pl.BlockSpec((_RC, 128), lambda i: (i + 4, 0)),
            pl.BlockSpec((128, 128), lambda i: (0, 0)),
        ],
        out_specs=[pl.BlockSpec((_RC, 128), lambda i: (i, 0))] * 2,
        out_shape=[jax.ShapeDtypeStruct((MROWS, 128), jnp.float32)] * 2,
    )(partsA, partsA, partsB, partsB, S)


_RF = MROWS // 23


def _final_body(xv_ref, ta_ref, tb_ref, pa0_ref, pa1_ref, pb0_ref, pb1_ref,
                s_ref, abd_ref, vmat_ref, dbd_ref, w3_ref, b3_ref, w4_ref,
                b4_ref, w5_ref, b5_ref, o_ref):
    f32 = jnp.float32
    dn = (((1,), (0,)), ((), ()))
    dot = lambda a, b: lax.dot_general(a, b, dn, preferred_element_type=f32)
    sa = pa0_ref[...] + pa1_ref[...]
    sb = pb0_ref[...] + pb1_ref[...]
    degr = dot(sb, s_ref[...])
    degb = jnp.maximum(degr, 1.0)
    vin = jnp.concatenate(
        [ta_ref[...], tb_ref[...], sa / degb, sb / degb,
         jnp.where(degr > 0.0, 1.0, 0.0)], axis=1)
    h2 = dot(xv_ref[...], abd_ref[...]) + dot(vin, vmat_ref[...]) + dbd_ref[0:1]
    h3 = jnp.maximum(dot(h2, w3_ref[...]) + b3_ref[0:1], 0.0)
    h4 = jnp.maximum(dot(h3, w4_ref[...]) + b4_ref[0:1], 0.0)
    o_ref[...] = jax.nn.sigmoid(dot(h4, w5_ref[...]) + b5_ref[0:1])


def _final(xv, tA, tB, pA, pB, S, abd, vmat, dbd, w3bd, b3bd, w4bd, b4bd,
           w5bd, b5bd):
    full = lambda shape: pl.BlockSpec(shape, lambda i: tuple(0 for _ in shape))
    mspec = lambda off: pl.BlockSpec((_RF, 128), lambda i, o=off: (i + o, 0))
    return pl.pallas_call(
        _final_body,
        grid=(MROWS // _RF,),
        in_specs=[
            pl.BlockSpec((_RF, 256), lambda i: (i, 0)),
            mspec(0),
            mspec(0),
            mspec(0),
            mspec(23),
            mspec(0),
            mspec(23),
            full((128, 128)),
            full((256, 512)),
            full((640, 512)),
            full((8, 512)),
            full((512, 256)),
            full((8, 256)),
            full((256, 128)),
            full((8, 128)),
            full((128, 8)),
            full((8, 8)),
        ],
        out_specs=pl.BlockSpec((_RF, 8), lambda i: (i, 0)),
        out_shape=jax.ShapeDtypeStruct((MROWS, 8), jnp.float32),
    )(xv, tA, tB, pA, pA, pB, pB, S, abd, vmat, dbd, w3bd, b3bd, w4bd, b4bd,
      w5bd, b5bd)


def kernel(x, edge_index, W1_self, W1_neigh, b1, W2_self, W2_neigh, b2,
           W3, b3, W4, b4, W5, b5):
    f32 = jnp.float32
    xv32 = jnp.pad(x, ((0, NP_ACC - N), (0, 15)))
    tabA = xv32[:, 0:16]
    tabB = xv32[:, 16:32] + jnp.zeros((1, 16), f32).at[0, 1].set(1.0)

    pad = jnp.full((EPAD,), N, jnp.int32)
    srcp = jnp.concatenate([edge_index[0], pad]).reshape(NBLKP, BLK)
    dstp = jnp.concatenate([edge_index[1], pad]).reshape(NBLKP, BLK)

    from jax.scipy.linalg import block_diag
    A = W1_self @ W2_self
    B = W1_neigh @ W2_self + W1_self @ W2_neigh
    C = W1_neigh @ W2_neigh
    d0 = b1 @ W2_self + b2
    c0 = b1 @ W2_neigh
    A32 = jnp.concatenate([A, jnp.zeros((15, 64), f32)], axis=0)
    row1 = lambda v: jnp.zeros((16, 64), f32).at[0].set(v)
    bd8 = lambda m: block_diag(*([m] * 8))
    abd = bd8(A32)
    vmat = jnp.concatenate(
        [bd8(B[:16]), bd8(row1(B[16])), bd8(C[:16]), bd8(row1(C[16])),
         bd8(row1(c0))], axis=0)
    rowp = lambda v, w: jnp.zeros((8, w), f32).at[0].set(jnp.tile(v, 8))
    dbd = rowp(d0, 512)
    w3bd, b3bd = bd8(W3), rowp(b3, 256)
    w4bd, b4bd = bd8(W4), rowp(b4, 128)
    w5bd, b5bd = bd8(W5), rowp(b5, 8)

    li = jnp.arange(128)
    S = ((li[:, None] % DW == 1) & (li[:, None] // DW == li[None, :] // DW)
         ).astype(f32)
    xv = xv32.reshape(MROWS, 256)

    partsA1 = _sc_segment_sum(tabA, srcp, dstp)
    partsB1 = _sc_segment_sum(tabB, srcp, dstp)
    t2A, t2B = _combine(partsA1, partsB1, S)
    partsA2 = _sc_segment_sum(t2A.reshape(NP_ACC, DW), srcp, dstp)
    partsB2 = _sc_segment_sum(t2B.reshape(NP_ACC, DW), srcp, dstp)
    out8 = _final(xv, t2A, t2B, partsA2, partsB2, S, abd, vmat, dbd,
                  w3bd, b3bd, w4bd, b4bd, w5bd, b5bd)
    return out8.reshape(NP_ACC, 1)[:N]


# speedup vs baseline: 1.1424x; 1.0387x over previous
import functools

import jax
import jax.numpy as jnp
from jax import lax
from jax.experimental import pallas as pl
from jax.experimental.pallas import tpu as pltpu
from jax.experimental.pallas import tpu_sc as plsc

N = 100000
E = 1600000
DW = 16
BLK = 128
NBLKP = 12512
EPAD = NBLKP * BLK - E
BLK_PER_TILE = NBLKP // 32
NP_ACC = 100096
ROWS_PER_TILE = NP_ACC // 16
CH = 13
NCH = 30


@functools.cache
def _sc_kernel():
    mesh = plsc.VectorSubcoreMesh(core_axis_name="c", subcore_axis_name="s",
                                  num_cores=2, num_subcores=16)
    return functools.partial(
        pl.kernel,
        out_type=jax.ShapeDtypeStruct((2 * NP_ACC, DW), jnp.float32),
        mesh=mesh,
        scratch_types=[
            pltpu.VMEM_SHARED((NP_ACC, DW), jnp.float32),
            pltpu.VMEM((CH, BLK), jnp.int32),
            pltpu.VMEM((CH, BLK), jnp.int32),
            pltpu.VMEM((CH, BLK, DW), jnp.float32),
            pltpu.SemaphoreType.DMA,
            pltpu.SemaphoreType.DMA,
        ],
        compiler_params=pltpu.CompilerParams(use_tc_tiling_on_sc=False),
    )(_sc_body)


def _sc_segment_sum(tab, srcb, dstb):
    return _sc_kernel()(tab, srcb, dstb).reshape(2 * NP_ACC * DW // 128, 128)


def _sc_body(tab, srcb, dstb, out, acc, sidx, didx, rows, gsem, ssem):
    c = lax.axis_index("c")
    s = lax.axis_index("s")
    wid = c * 16 + s

    z16 = jnp.zeros((16,), jnp.float32)

    def _zb(i, carry):
        rows[0, i, pl.ds(0, 16)] = z16
        return carry

    lax.fori_loop(0, BLK, _zb, 0)
    zbase = s * ROWS_PER_TILE
    for k in range(ROWS_PER_TILE // BLK):
        pltpu.sync_copy(rows.at[0], acc.at[pl.ds(zbase + k * BLK, BLK)])
    pltpu.sync_copy(rows.at[0], acc.at[pl.ds(zbase + ROWS_PER_TILE - BLK, BLK)])
    plsc.subcore_barrier()

    start = wid * BLK_PER_TILE

    def _chunk(t, carry):
        base = start + t * CH
        pltpu.sync_copy(srcb.at[pl.ds(base, CH)], sidx)
        pltpu.sync_copy(dstb.at[pl.ds(base, CH)], didx)
        gd = [pltpu.async_copy(tab.at[sidx.at[j]], rows.at[j], gsem)
              for j in range(CH)]
        for d in gd:
            d.wait()
        sd = [pltpu.async_copy(rows.at[j], acc.at[didx.at[j]], ssem, add=True)
              for j in range(CH)]
        for d in sd:
            d.wait()
        return carry

    lax.fori_loop(0, NCH, _chunk, 0)
    tbase = start + CH * NCH
    pltpu.sync_copy(srcb.at[pl.ds(tbase, 1)], sidx.at[pl.ds(0, 1)])
    pltpu.sync_copy(dstb.at[pl.ds(tbase, 1)], didx.at[pl.ds(0, 1)])
    pltpu.async_copy(tab.at[sidx.at[0]], rows.at[0], gsem).wait()
    pltpu.sync_copy(rows.at[0], acc.at[didx.at[0]], add=True)
    plsc.subcore_barrier()

    pltpu.sync_copy(acc.at[pl.ds(zbase, ROWS_PER_TILE)],
                    out.at[pl.ds(c * NP_ACC + zbase, ROWS_PER_TILE)])


MROWS = NP_ACC * DW // 128
_RC = MROWS // 4


def _combine_body(pa0_ref, pa1_ref, pb0_ref, pb1_ref, s_ref, oa_ref, ob_ref):
    sa = pa0_ref[...] + pa1_ref[...]
    sb = pb0_ref[...] + pb1_ref[...]
    degb = jnp.maximum(
        lax.dot_general(sb, s_ref[...], (((1,), (0,)), ((), ())),
                        preferred_element_type=jnp.float32), 1.0)
    oa_ref[...] = sa / degb
    lane = lax.broadcasted_iota(jnp.int32, (_RC, 128), 1) % DW
    ob_ref[...] = jnp.where(lane == 0, sb / degb,
                            jnp.where(lane == 1, 1.0, 0.0))


def _combine(partsA, partsB, S):
    return pl.pallas_call(
        _combine_body,
        grid=(4,),
        in_specs=[
            pl.BlockSpec((_RC, 128), lambda i: (i, 0)),
            pl.BlockSpec((_RC, 128), lambda i: (i + 4, 0)),
            pl.BlockSpec((_RC, 128), lambda i: (i, 0)),
            pl.BlockSpec((_RC, 128), lambda i: (i + 4, 0)),
            pl.BlockSpec((128, 128), lambda i: (0, 0)),
        ],
        out_specs=[pl.BlockSpec((_RC, 128), lambda i: (i, 0))] * 2,
        out_shape=[jax.ShapeDtypeStruct((MROWS, 128), jnp.float32)] * 2,
    )(partsA, partsA, partsB, partsB, S)


_RF = MROWS // 23


def _tabprep_body(xv_ref, pa_ref, pb_ref, ov_ref, oa_ref, ob_ref):
    dn = (((1,), (0,)), ((), ()))
    dot = lambda a, b: lax.dot_general(a, b, dn,
                                       preferred_element_type=jnp.float32)
    oa_ref[...] = dot(xv_ref[...], pa_ref[...])
    ob_ref[...] = dot(xv_ref[...], pb_ref[...]) + ov_ref[0:1]


def _tabprep(xv, PA, PB, onesv):
    full = lambda shape: pl.BlockSpec(shape, lambda i: tuple(0 for _ in shape))
    return pl.pallas_call(
        _tabprep_body,
        grid=(MROWS // _RC,),
        in_specs=[
            pl.BlockSpec((_RC, 256), lambda i: (i, 0)),
            full((256, 128)),
            full((256, 128)),
            full((8, 128)),
        ],
        out_specs=[pl.BlockSpec((_RC, 128), lambda i: (i, 0))] * 2,
        out_shape=[jax.ShapeDtypeStruct((MROWS, 128), jnp.float32)] * 2,
    )(xv, PA, PB, onesv)


def _final_body(xv_ref, ta_ref, tb_ref, pa0_ref, pa1_ref, pb0_ref, pb1_ref,
                s_ref, abd_ref, vmat_ref, dbd_ref, w3_ref, b3_ref, w4_ref,
                b4_ref, w5_ref, b5_ref, o_ref):
    f32 = jnp.float32
    dn = (((1,), (0,)), ((), ()))
    dot = lambda a, b: lax.dot_general(a, b, dn, preferred_element_type=f32)
    sa = pa0_ref[...] + pa1_ref[...]
    sb = pb0_ref[...] + pb1_ref[...]
    degr = dot(sb, s_ref[...])
    degb = jnp.maximum(degr, 1.0)
    vin = jnp.concatenate(
        [ta_ref[...], tb_ref[...], sa / degb, sb / degb,
         jnp.where(degr > 0.0, 1.0, 0.0)], axis=1)
    h2 = dot(xv_ref[...], abd_ref[...]) + dot(vin, vmat_ref[...]) + dbd_ref[0:1]
    h3 = jnp.maximum(dot(h2, w3_ref[...]) + b3_ref[0:1], 0.0)
    h4 = jnp.maximum(dot(h3, w4_ref[...]) + b4_ref[0:1], 0.0)
    o_ref[...] = jax.nn.sigmoid(dot(h4, w5_ref[...]) + b5_ref[0:1])


def _final(xv, tA, tB, pA, pB, S, abd, vmat, dbd, w3bd, b3bd, w4bd, b4bd,
           w5bd, b5bd):
    full = lambda shape: pl.BlockSpec(shape, lambda i: tuple(0 for _ in shape))
    mspec = lambda off: pl.BlockSpec((_RF, 128), lambda i, o=off: (i + o, 0))
    return pl.pallas_call(
        _final_body,
        grid=(MROWS // _RF,),
        in_specs=[
            pl.BlockSpec((_RF, 256), lambda i: (i, 0)),
            mspec(0),
            mspec(0),
            mspec(0),
            mspec(23),
            mspec(0),
            mspec(23),
            full((128, 128)),
            full((256, 512)),
            full((640, 512)),
            full((8, 512)),
            full((512, 256)),
            full((8, 256)),
            full((256, 128)),
            full((8, 128)),
            full((128, 8)),
            full((8, 8)),
        ],
        out_specs=pl.BlockSpec((_RF, 8), lambda i: (i, 0)),
        out_shape=jax.ShapeDtypeStruct((MROWS, 8), jnp.float32),
    )(xv, tA, tB, pA, pA, pB, pB, S, abd, vmat, dbd, w3bd, b3bd, w4bd, b4bd,
      w5bd, b5bd)


def kernel(x, edge_index, W1_self, W1_neigh, b1, W2_self, W2_neigh, b2,
           W3, b3, W4, b4, W5, b5):
    f32 = jnp.float32
    xv = jnp.pad(x, ((0, NP_ACC - N), (0, 15))).reshape(MROWS, 256)
    g = jnp.arange(8).repeat(16)
    k = jnp.tile(jnp.arange(16), 8)
    PA = jnp.zeros((256, 128), f32).at[32 * g + k, 16 * g + k].set(1.0)
    PB = jnp.zeros((256, 128), f32).at[32 * g + 16 + k, 16 * g + k].set(1.0)
    onesv = jnp.zeros((8, 128), f32).at[0, 16 * jnp.arange(8) + 1].set(1.0)

    pad = jnp.full((EPAD,), N, jnp.int32)
    srcp = jnp.concatenate([edge_index[0], pad]).reshape(NBLKP, BLK)
    dstp = jnp.concatenate([edge_index[1], pad]).reshape(NBLKP, BLK)

    from jax.scipy.linalg import block_diag
    A = W1_self @ W2_self
    B = W1_neigh @ W2_self + W1_self @ W2_neigh
    C = W1_neigh @ W2_neigh
    d0 = b1 @ W2_self + b2
    c0 = b1 @ W2_neigh
    A32 = jnp.concatenate([A, jnp.zeros((15, 64), f32)], axis=0)
    row1 = lambda v: jnp.zeros((16, 64), f32).at[0].set(v)
    bd8 = lambda m: block_diag(*([m] * 8))
    abd = bd8(A32)
    vmat = jnp.concatenate(
        [bd8(B[:16]), bd8(row1(B[16])), bd8(C[:16]), bd8(row1(C[16])),
         bd8(row1(c0))], axis=0)
    rowp = lambda v, w: jnp.zeros((8, w), f32).at[0].set(jnp.tile(v, 8))
    dbd = rowp(d0, 512)
    w3bd, b3bd = bd8(W3), rowp(b3, 256)
    w4bd, b4bd = bd8(W4), rowp(b4, 128)
    w5bd, b5bd = bd8(W5), rowp(b5, 8)

    li = jnp.arange(128)
    S = ((li[:, None] % DW == 1) & (li[:, None] // DW == li[None, :] // DW)
         ).astype(f32)
    tAv, tBv = _tabprep(xv, PA, PB, onesv)
    partsA1 = _sc_segment_sum(tAv.reshape(NP_ACC, DW), srcp, dstp)
    partsB1 = _sc_segment_sum(tBv.reshape(NP_ACC, DW), srcp, dstp)
    t2A, t2B = _combine(partsA1, partsB1, S)
    partsA2 = _sc_segment_sum(t2A.reshape(NP_ACC, DW), srcp, dstp)
    partsB2 = _sc_segment_sum(t2B.reshape(NP_ACC, DW), srcp, dstp)
    out8 = _final(xv, t2A, t2B, partsA2, partsB2, S, abd, vmat, dbd,
                  w3bd, b3bd, w4bd, b4bd, w5bd, b5bd)
    return out8.reshape(NP_ACC, 1)[:N]
